# baseline (speedup 1.0000x reference)
import functools

import jax
import jax.numpy as jnp
from jax import lax
from jax.experimental import pallas as pl
from jax.experimental.pallas import tpu as pltpu
from jax.experimental.pallas import tpu_sc as plsc

_NC, _NS = 2, 16
_NW = _NC * _NS

_ROWS, _D = 32768, 2048
_ROWS_PER_W = _ROWS // _NW
_CHUNK = 32
_NCHUNKS = _ROWS_PER_W // _CHUNK


def _sc_copy(x_hbm, out_hbm, buf):
    wid = lax.axis_index("s") * _NC + lax.axis_index("c")
    base = wid * _ROWS_PER_W

    def body(j, _):
        off = base + j * _CHUNK
        pltpu.sync_copy(x_hbm.at[pl.ds(off, _CHUNK)], buf)
        pltpu.sync_copy(buf, out_hbm.at[pl.ds(off, _CHUNK)])
        return ()

    lax.fori_loop(0, _NCHUNKS, body, (), unroll=False)


def kernel(hidden_state):
    B, S, D = hidden_state.shape
    x = hidden_state.reshape(B * S, D)
    mesh = plsc.VectorSubcoreMesh(core_axis_name="c", subcore_axis_name="s")
    k = functools.partial(
        pl.kernel,
        mesh=mesh,
        out_type=jax.ShapeDtypeStruct((B * S, D), hidden_state.dtype),
        scratch_types=[pltpu.VMEM((_CHUNK, D), hidden_state.dtype)],
    )(_sc_copy)
    out = k(x)
    return out.reshape(B, S, D)


# speedup vs baseline: 1.0504x; 1.0504x over previous
import functools

import jax
import jax.numpy as jnp
from jax import lax
from jax.experimental import pallas as pl
from jax.experimental.pallas import tpu as pltpu
from jax.experimental.pallas import tpu_sc as plsc

_NC, _NS = 2, 16
_NW = _NC * _NS

_ROWS, _D = 32768, 2048
_ROWS_PER_W = _ROWS // _NW
_CHUNK = 16
_NCHUNKS = _ROWS_PER_W // _CHUNK


def _sc_copy(x_hbm, out_hbm, buf_a, buf_b, sem):
    wid = lax.axis_index("s") * _NC + lax.axis_index("c")
    base = wid * _ROWS_PER_W

    def read(j, b):
        return pltpu.async_copy(x_hbm.at[pl.ds(base + j * _CHUNK, _CHUNK)], b, sem)

    handle = read(0, buf_a)
    for j in range(_NCHUNKS):
        cur, nxt = (buf_a, buf_b) if j % 2 == 0 else (buf_b, buf_a)
        handle.wait()
        if j + 1 < _NCHUNKS:
            handle = read(j + 1, nxt)
        pltpu.sync_copy(cur, out_hbm.at[pl.ds(base + j * _CHUNK, _CHUNK)])


def kernel(hidden_state):
    B, S, D = hidden_state.shape
    x = hidden_state.reshape(B * S, D)
    mesh = plsc.VectorSubcoreMesh(core_axis_name="c", subcore_axis_name="s")
    k = functools.partial(
        pl.kernel,
        mesh=mesh,
        out_type=jax.ShapeDtypeStruct((B * S, D), hidden_state.dtype),
        scratch_types=[
            pltpu.VMEM((_CHUNK, D), hidden_state.dtype),
            pltpu.VMEM((_CHUNK, D), hidden_state.dtype),
            pltpu.SemaphoreType.DMA,
        ],
    )(_sc_copy)
    out = k(x)
    return out.reshape(B, S, D)


# speedup vs baseline: 1.3319x; 1.2680x over previous
import jax
import jax.numpy as jnp
from jax.experimental import pallas as pl
from jax.experimental.pallas import tpu as pltpu


def _copy_body(x_ref, o_ref):
    o_ref[...] = x_ref[...]


def kernel(hidden_state):
    B, S, D = hidden_state.shape
    x = hidden_state.reshape(B * S, D)
    rows = B * S
    block_rows = 2040
    grid = (pl.cdiv(rows, block_rows),)
    out = pl.pallas_call(
        _copy_body,
        grid=grid,
        in_specs=[pl.BlockSpec((block_rows, D), lambda i: (i, 0))],
        out_specs=pl.BlockSpec((block_rows, D), lambda i: (i, 0)),
        out_shape=jax.ShapeDtypeStruct((rows, D), hidden_state.dtype),
        compiler_params=pltpu.CompilerParams(
            vmem_limit_bytes=100 * 1024 * 1024,
        ),
    )(x)
    return out.reshape(B, S, D)
